# Initial kernel scaffold; baseline (speedup 1.0000x reference)
import jax
import jax.numpy as jnp
from jax.experimental import pallas as pl


def kernel(x, edge_index, W1l, b1l, W1r, W2l, b2l, W2r):
    raise NotImplementedError("write your pallas kernel here")


# speedup vs baseline: 7.5829x; 7.5829x over previous
import functools

import jax
import jax.numpy as jnp
from jax import lax
from jax.experimental import pallas as pl
from jax.experimental.pallas import tpu as pltpu
from jax.experimental.pallas import tpu_sc as plsc

N = 10000
NP = 10240
D = 128
DH = D // 2
E = 320000
NC, NS, L = 2, 16, 16
NT = NC * NS
CHUNK = 80
CPT = (E // NS) // CHUNK
CPT2 = (E // NT) // CHUNK
RPT = NP // NS
ZR = 128
BN = 1280

_mesh = plsc.VectorSubcoreMesh(core_axis_name="c", subcore_axis_name="s")


@functools.partial(
    pl.kernel,
    out_type=(
        jax.ShapeDtypeStruct((NC, NP, DH), jnp.float32),
        jax.ShapeDtypeStruct((NP,), jnp.float32),
    ),
    mesh=_mesh,
    scratch_types=[
        pltpu.VMEM((CPT, CHUNK), jnp.int32),
        pltpu.VMEM((CPT, CHUNK), jnp.int32),
        pltpu.VMEM((CHUNK, DH), jnp.float32),
        pltpu.VMEM((CHUNK,), jnp.float32),
        pltpu.VMEM((ZR, DH), jnp.float32),
        pltpu.VMEM((RPT,), jnp.float32),
        pltpu.VMEM_SHARED((NP, DH), jnp.float32),
        pltpu.VMEM_SHARED((NP,), jnp.float32),
        pltpu.SemaphoreType.DMA,
    ],
    compiler_params=pltpu.CompilerParams(use_tc_tiling_on_sc=False),
)
def _sc_agg_rows(x0_hbm, x1_hbm, src_hbm, dst_hbm, acc_out, deg_out,
                 src_buf, dst_buf, rows_v, ones_v, zrow, zdeg,
                 acc_sh, deg_sh, sem):
    cid = lax.axis_index("c")
    sid = lax.axis_index("s")

    def zfill(r, _):
        for k in range(DH // L):
            zrow[r, pl.ds(k * L, L)] = jnp.zeros((L,), jnp.float32)
        return 0
    lax.fori_loop(0, ZR, zfill, 0)
    for k in range(RPT // L):
        zdeg[pl.ds(k * L, L)] = jnp.zeros((L,), jnp.float32)
    for k in range(CHUNK // L):
        ones_v[pl.ds(k * L, L)] = jnp.ones((L,), jnp.float32)

    for k in range(RPT // ZR):
        pltpu.sync_copy(zrow, acc_sh.at[pl.ds(sid * RPT + k * ZR, ZR)])

    @pl.when(cid == 0)
    def _():
        pltpu.sync_copy(zdeg, deg_sh.at[pl.ds(sid * RPT, RPT)])
    plsc.subcore_barrier()

    pltpu.sync_copy(src_hbm.at[sid], src_buf)
    pltpu.sync_copy(dst_hbm.at[sid], dst_buf)

    def body(j, _):
        @pl.when(cid == 0)
        def _():
            pltpu.async_copy(x0_hbm.at[src_buf.at[j]], rows_v, sem).wait()

        @pl.when(cid == 1)
        def _():
            pltpu.async_copy(x1_hbm.at[src_buf.at[j]], rows_v, sem).wait()

        pltpu.sync_copy(rows_v, acc_sh.at[dst_buf.at[j]], add=True)

        @pl.when(cid == 0)
        def _():
            pltpu.sync_copy(ones_v, deg_sh.at[dst_buf.at[j]], add=True)
        return 0
    lax.fori_loop(0, CPT, body, 0)
    plsc.subcore_barrier()

    pltpu.sync_copy(acc_sh.at[pl.ds(sid * RPT, RPT)],
                    acc_out.at[cid, pl.ds(sid * RPT, RPT)])

    @pl.when(cid == 0)
    def _():
        pltpu.sync_copy(deg_sh.at[pl.ds(sid * RPT, RPT)],
                        deg_out.at[pl.ds(sid * RPT, RPT)])


@functools.partial(
    pl.kernel,
    out_type=jax.ShapeDtypeStruct((NC, NP), jnp.float32),
    mesh=_mesh,
    scratch_types=[
        pltpu.VMEM((CPT2, CHUNK), jnp.int32),
        pltpu.VMEM((CPT2, CHUNK), jnp.int32),
        pltpu.VMEM((CHUNK,), jnp.float32),
        pltpu.VMEM((RPT,), jnp.float32),
        pltpu.VMEM_SHARED((NP,), jnp.float32),
        pltpu.SemaphoreType.DMA,
    ],
    compiler_params=pltpu.CompilerParams(use_tc_tiling_on_sc=False),
)
def _sc_agg_scalar(s_hbm, src_hbm, dst_hbm, agg_out,
                   src_buf, dst_buf, vals_v, zdeg, agg_sh, sem):
    cid = lax.axis_index("c")
    sid = lax.axis_index("s")
    w = cid * NS + sid

    for k in range(RPT // L):
        zdeg[pl.ds(k * L, L)] = jnp.zeros((L,), jnp.float32)
    pltpu.sync_copy(zdeg, agg_sh.at[pl.ds(sid * RPT, RPT)])
    plsc.subcore_barrier()

    pltpu.sync_copy(src_hbm.at[w], src_buf)
    pltpu.sync_copy(dst_hbm.at[w], dst_buf)

    def body(j, _):
        pltpu.async_copy(s_hbm.at[src_buf.at[j]], vals_v, sem).wait()
        pltpu.sync_copy(vals_v, agg_sh.at[dst_buf.at[j]], add=True)
        return 0
    lax.fori_loop(0, CPT2, body, 0)
    plsc.subcore_barrier()

    pltpu.sync_copy(agg_sh.at[pl.ds(sid * RPT, RPT)],
                    agg_out.at[cid, pl.ds(sid * RPT, RPT)])


def _tc_dense_body(acc_ref, deg_ref, x_ref, w1lt_ref, b1l_ref, w1rt_ref,
                   w2lt_ref, w2rt_ref, s2l_ref, s2r_ref):
    d = jnp.maximum(deg_ref[...], 1.0)
    m0 = acc_ref[0] / d
    m1 = acc_ref[1] / d
    w1lt = w1lt_ref[...]
    h = (jnp.dot(m0, w1lt[:DH], preferred_element_type=jnp.float32)
         + jnp.dot(m1, w1lt[DH:], preferred_element_type=jnp.float32)
         + b1l_ref[...]
         + jnp.dot(x_ref[...], w1rt_ref[...], preferred_element_type=jnp.float32))
    h = jnp.maximum(h, 0.0)
    s2l_ref[...] = jnp.dot(h, w2lt_ref[...], preferred_element_type=jnp.float32)
    s2r_ref[...] = jnp.dot(h, w2rt_ref[...], preferred_element_type=jnp.float32)


_tc_dense = pl.pallas_call(
    _tc_dense_body,
    grid=(NP // BN,),
    in_specs=[
        pl.BlockSpec((NC, BN, DH), lambda i: (0, i, 0)),
        pl.BlockSpec((BN, 1), lambda i: (i, 0)),
        pl.BlockSpec((BN, D), lambda i: (i, 0)),
        pl.BlockSpec((D, D), lambda i: (0, 0)),
        pl.BlockSpec((1, D), lambda i: (0, 0)),
        pl.BlockSpec((D, D), lambda i: (0, 0)),
        pl.BlockSpec((D, 1), lambda i: (0, 0)),
        pl.BlockSpec((D, 1), lambda i: (0, 0)),
    ],
    out_specs=[
        pl.BlockSpec((BN, 1), lambda i: (i, 0)),
        pl.BlockSpec((BN, 1), lambda i: (i, 0)),
    ],
    out_shape=[
        jax.ShapeDtypeStruct((NP, 1), jnp.float32),
        jax.ShapeDtypeStruct((NP, 1), jnp.float32),
    ],
)


def _tc_final_body(agg_ref, deg_ref, s2r_ref, b2l_ref, out_ref):
    a = agg_ref[0] + agg_ref[1]
    d = jnp.maximum(deg_ref[...], 1.0)
    out_ref[...] = a / d + b2l_ref[...] + s2r_ref[...]


_tc_final = pl.pallas_call(
    _tc_final_body,
    grid=(NP // BN,),
    in_specs=[
        pl.BlockSpec((NC, BN, 1), lambda i: (0, i, 0)),
        pl.BlockSpec((BN, 1), lambda i: (i, 0)),
        pl.BlockSpec((BN, 1), lambda i: (i, 0)),
        pl.BlockSpec((1, 1), lambda i: (0, 0)),
    ],
    out_specs=pl.BlockSpec((BN, 1), lambda i: (i, 0)),
    out_shape=jax.ShapeDtypeStruct((NP, 1), jnp.float32),
)


def kernel(x, edge_index, W1l, b1l, W1r, W2l, b2l, W2r):
    src_a = edge_index[0].reshape(NS, CPT, CHUNK)
    dst_a = edge_index[1].reshape(NS, CPT, CHUNK)
    src_b = edge_index[0].reshape(NT, CPT2, CHUNK)
    dst_b = edge_index[1].reshape(NT, CPT2, CHUNK)

    x0 = x[:, :DH]
    x1 = x[:, DH:]
    acc, deg = _sc_agg_rows(x0, x1, src_a, dst_a)
    deg2 = deg.reshape(NP, 1)

    x_pad = jnp.zeros((NP, D), jnp.float32).at[:N].set(x)
    s2l, s2r = _tc_dense(acc, deg2, x_pad, W1l.T, b1l.reshape(1, D), W1r.T,
                         W2l.T, W2r.T)

    agg2 = _sc_agg_scalar(s2l.reshape(NP), src_b, dst_b)
    out = _tc_final(agg2.reshape(NC, NP, 1), deg2, s2r, b2l.reshape(1, 1))
    return out[:N, 0]


# speedup vs baseline: 13.5984x; 1.7933x over previous
import functools

import jax
import jax.numpy as jnp
from jax import lax
from jax.experimental import pallas as pl
from jax.experimental.pallas import tpu as pltpu
from jax.experimental.pallas import tpu_sc as plsc

N = 10000
NP = 10240
D = 128
DH = D // 2
E = 320000
NC, NS, L = 2, 16, 16
NT = NC * NS
CHUNK = 80
CPT = (E // NS) // CHUNK
CPT2 = (E // NT) // CHUNK
RPT = NP // NS
ZR = 128
BN = 1280

_mesh = plsc.VectorSubcoreMesh(core_axis_name="c", subcore_axis_name="s")


@functools.partial(
    pl.kernel,
    out_type=(
        jax.ShapeDtypeStruct((NC, NP, DH), jnp.float32),
        jax.ShapeDtypeStruct((NP,), jnp.float32),
    ),
    mesh=_mesh,
    scratch_types=[
        pltpu.VMEM((CPT, CHUNK), jnp.int32),
        pltpu.VMEM((CPT, CHUNK), jnp.int32),
        pltpu.VMEM((CHUNK, DH), jnp.float32),
        pltpu.VMEM((CHUNK, DH), jnp.float32),
        pltpu.VMEM((CHUNK,), jnp.float32),
        pltpu.VMEM((ZR, DH), jnp.float32),
        pltpu.VMEM((RPT,), jnp.float32),
        pltpu.VMEM_SHARED((NP, DH), jnp.float32),
        pltpu.VMEM_SHARED((NP,), jnp.float32),
        pltpu.SemaphoreType.DMA,
        pltpu.SemaphoreType.DMA,
    ],
    compiler_params=pltpu.CompilerParams(use_tc_tiling_on_sc=False),
)
def _sc_agg_rows(x0_hbm, x1_hbm, src_hbm, dst_hbm, acc_out, deg_out,
                 src_buf, dst_buf, rows_a, rows_b, ones_v, zrow, zdeg,
                 acc_sh, deg_sh, sem_a, sem_b):
    cid = lax.axis_index("c")
    sid = lax.axis_index("s")

    def zfill(r, _):
        for k in range(DH // L):
            zrow[r, pl.ds(k * L, L)] = jnp.zeros((L,), jnp.float32)
        return 0
    lax.fori_loop(0, ZR, zfill, 0)
    for k in range(RPT // L):
        zdeg[pl.ds(k * L, L)] = jnp.zeros((L,), jnp.float32)
    for k in range(CHUNK // L):
        ones_v[pl.ds(k * L, L)] = jnp.ones((L,), jnp.float32)

    for k in range(RPT // ZR):
        pltpu.sync_copy(zrow, acc_sh.at[pl.ds(sid * RPT + k * ZR, ZR)])

    @pl.when(cid == 0)
    def _():
        pltpu.sync_copy(zdeg, deg_sh.at[pl.ds(sid * RPT, RPT)])
    plsc.subcore_barrier()

    pltpu.sync_copy(src_hbm.at[sid], src_buf)
    pltpu.sync_copy(dst_hbm.at[sid], dst_buf)

    def edge_loop(x_hbm, with_deg):
        def start(j, buf, sem):
            pltpu.async_copy(x_hbm.at[src_buf.at[j]], buf, sem)

        def finish(j, buf, sem):
            pltpu.make_async_copy(x_hbm.at[src_buf.at[j]], buf, sem).wait()

        def consume(j, buf):
            pltpu.sync_copy(buf, acc_sh.at[dst_buf.at[j]], add=True)
            if with_deg:
                pltpu.sync_copy(ones_v, deg_sh.at[dst_buf.at[j]], add=True)

        start(0, rows_a, sem_a)

        def body2(i, _):
            j0 = 2 * i
            start(j0 + 1, rows_b, sem_b)
            finish(j0, rows_a, sem_a)
            consume(j0, rows_a)

            @pl.when(j0 + 2 < CPT)
            def _():
                start(j0 + 2, rows_a, sem_a)
            finish(j0 + 1, rows_b, sem_b)
            consume(j0 + 1, rows_b)
            return 0
        lax.fori_loop(0, CPT // 2, body2, 0)

    @pl.when(cid == 0)
    def _():
        edge_loop(x0_hbm, True)

    @pl.when(cid == 1)
    def _():
        edge_loop(x1_hbm, False)
    plsc.subcore_barrier()

    pltpu.sync_copy(acc_sh.at[pl.ds(sid * RPT, RPT)],
                    acc_out.at[cid, pl.ds(sid * RPT, RPT)])

    @pl.when(cid == 0)
    def _():
        pltpu.sync_copy(deg_sh.at[pl.ds(sid * RPT, RPT)],
                        deg_out.at[pl.ds(sid * RPT, RPT)])


@functools.partial(
    pl.kernel,
    out_type=jax.ShapeDtypeStruct((NC, NP), jnp.float32),
    mesh=_mesh,
    scratch_types=[
        pltpu.VMEM((CPT2, CHUNK), jnp.int32),
        pltpu.VMEM((CPT2, CHUNK), jnp.int32),
        pltpu.VMEM((NP,), jnp.float32),
        pltpu.VMEM((CPT2, CHUNK), jnp.float32),
        pltpu.VMEM((RPT,), jnp.float32),
        pltpu.VMEM_SHARED((NP,), jnp.float32),
        pltpu.SemaphoreType.DMA,
    ],
    compiler_params=pltpu.CompilerParams(use_tc_tiling_on_sc=False,
                                         needs_layout_passes=False),
)
def _sc_agg_scalar(s_hbm, src_hbm, dst_hbm, agg_out,
                   src_buf, dst_buf, s_tile, vals_all, zdeg, agg_sh, sem):
    cid = lax.axis_index("c")
    sid = lax.axis_index("s")
    w = cid * NS + sid

    for k in range(RPT // L):
        zdeg[pl.ds(k * L, L)] = jnp.zeros((L,), jnp.float32)
    pltpu.sync_copy(zdeg, agg_sh.at[pl.ds(sid * RPT, RPT)])
    plsc.subcore_barrier()

    pltpu.sync_copy(s_hbm, s_tile)
    pltpu.sync_copy(src_hbm.at[w], src_buf)
    pltpu.sync_copy(dst_hbm.at[w], dst_buf)

    def gbody(j, _):
        for k in range(CHUNK // L):
            idx = src_buf[j, pl.ds(k * L, L)]
            vals_all[j, pl.ds(k * L, L)] = plsc.load_gather(s_tile, [idx])
        return 0
    lax.fori_loop(0, CPT2, gbody, 0)

    def sbody(j, _):
        pltpu.sync_copy(vals_all.at[j], agg_sh.at[dst_buf.at[j]], add=True)
        return 0
    lax.fori_loop(0, CPT2, sbody, 0)
    plsc.subcore_barrier()

    pltpu.sync_copy(agg_sh.at[pl.ds(sid * RPT, RPT)],
                    agg_out.at[cid, pl.ds(sid * RPT, RPT)])


def _tc_dense_body(acc_ref, deg_ref, x_ref, w1lt_ref, b1l_ref, w1rt_ref,
                   w2lt_ref, w2rt_ref, s2l_ref, s2r_ref):
    d = jnp.maximum(deg_ref[...], 1.0)
    m0 = acc_ref[0] / d
    m1 = acc_ref[1] / d
    w1lt = w1lt_ref[...]
    h = (jnp.dot(m0, w1lt[:DH], preferred_element_type=jnp.float32)
         + jnp.dot(m1, w1lt[DH:], preferred_element_type=jnp.float32)
         + b1l_ref[...]
         + jnp.dot(x_ref[...], w1rt_ref[...], preferred_element_type=jnp.float32))
    h = jnp.maximum(h, 0.0)
    s2l_ref[...] = jnp.dot(h, w2lt_ref[...], preferred_element_type=jnp.float32)
    s2r_ref[...] = jnp.dot(h, w2rt_ref[...], preferred_element_type=jnp.float32)


_tc_dense = pl.pallas_call(
    _tc_dense_body,
    grid=(NP // BN,),
    in_specs=[
        pl.BlockSpec((NC, BN, DH), lambda i: (0, i, 0)),
        pl.BlockSpec((BN, 1), lambda i: (i, 0)),
        pl.BlockSpec((BN, D), lambda i: (i, 0)),
        pl.BlockSpec((D, D), lambda i: (0, 0)),
        pl.BlockSpec((1, D), lambda i: (0, 0)),
        pl.BlockSpec((D, D), lambda i: (0, 0)),
        pl.BlockSpec((D, 1), lambda i: (0, 0)),
        pl.BlockSpec((D, 1), lambda i: (0, 0)),
    ],
    out_specs=[
        pl.BlockSpec((BN, 1), lambda i: (i, 0)),
        pl.BlockSpec((BN, 1), lambda i: (i, 0)),
    ],
    out_shape=[
        jax.ShapeDtypeStruct((NP, 1), jnp.float32),
        jax.ShapeDtypeStruct((NP, 1), jnp.float32),
    ],
)


def _tc_final_body(agg_ref, deg_ref, s2r_ref, b2l_ref, out_ref):
    a = agg_ref[0] + agg_ref[1]
    d = jnp.maximum(deg_ref[...], 1.0)
    out_ref[...] = a / d + b2l_ref[...] + s2r_ref[...]


_tc_final = pl.pallas_call(
    _tc_final_body,
    grid=(NP // BN,),
    in_specs=[
        pl.BlockSpec((NC, BN, 1), lambda i: (0, i, 0)),
        pl.BlockSpec((BN, 1), lambda i: (i, 0)),
        pl.BlockSpec((BN, 1), lambda i: (i, 0)),
        pl.BlockSpec((1, 1), lambda i: (0, 0)),
    ],
    out_specs=pl.BlockSpec((BN, 1), lambda i: (i, 0)),
    out_shape=jax.ShapeDtypeStruct((NP, 1), jnp.float32),
)


def kernel(x, edge_index, W1l, b1l, W1r, W2l, b2l, W2r):
    src_a = edge_index[0].reshape(NS, CPT, CHUNK)
    dst_a = edge_index[1].reshape(NS, CPT, CHUNK)
    src_b = edge_index[0].reshape(NT, CPT2, CHUNK)
    dst_b = edge_index[1].reshape(NT, CPT2, CHUNK)

    x0 = x[:, :DH]
    x1 = x[:, DH:]
    acc, deg = _sc_agg_rows(x0, x1, src_a, dst_a)
    deg2 = deg.reshape(NP, 1)

    x_pad = jnp.zeros((NP, D), jnp.float32).at[:N].set(x)
    s2l, s2r = _tc_dense(acc, deg2, x_pad, W1l.T, b1l.reshape(1, D), W1r.T,
                         W2l.T, W2r.T)

    agg2 = _sc_agg_scalar(s2l.reshape(NP), src_b, dst_b)
    out = _tc_final(agg2.reshape(NC, NP, 1), deg2, s2r, b2l.reshape(1, 1))
    return out[:N, 0]


# speedup vs baseline: 14.5009x; 1.0664x over previous
import functools

import jax
import jax.numpy as jnp
from jax import lax
from jax.experimental import pallas as pl
from jax.experimental.pallas import tpu as pltpu
from jax.experimental.pallas import tpu_sc as plsc

N = 10000
NP = 10240
D = 128
DH = D // 2
E = 320000
NC, NS, L = 2, 16, 16
NT = NC * NS
CH1 = 125
CPT1 = (E // NS) // CH1
CH2 = 80
CPT2 = (E // NT) // CH2
RPT = NP // NS
ZR = 128
BN = 2000

_mesh = plsc.VectorSubcoreMesh(core_axis_name="c", subcore_axis_name="s")


@functools.partial(
    pl.kernel,
    out_type=(
        jax.ShapeDtypeStruct((NC, NP, DH), jnp.float32),
        jax.ShapeDtypeStruct((NP,), jnp.float32),
    ),
    mesh=_mesh,
    scratch_types=[
        pltpu.VMEM((CPT1, CH1), jnp.int32),
        pltpu.VMEM((CPT1, CH1), jnp.int32),
        pltpu.VMEM((CH1, DH), jnp.float32),
        pltpu.VMEM((CH1, DH), jnp.float32),
        pltpu.VMEM((ZR,), jnp.float32),
        pltpu.VMEM((ZR, DH), jnp.float32),
        pltpu.VMEM((RPT,), jnp.float32),
        pltpu.VMEM_SHARED((NP, DH), jnp.float32),
        pltpu.VMEM_SHARED((NP,), jnp.float32),
        pltpu.SemaphoreType.DMA,
        pltpu.SemaphoreType.DMA,
    ],
    compiler_params=pltpu.CompilerParams(use_tc_tiling_on_sc=False),
)
def _sc_agg_rows(x0_hbm, x1_hbm, src_hbm, dst_hbm, acc_out, deg_out,
                 src_buf, dst_buf, rows_a, rows_b, ones_v, zrow, zdeg,
                 acc_sh, deg_sh, sem_a, sem_b):
    cid = lax.axis_index("c")
    sid = lax.axis_index("s")

    def zfill(r, _):
        for k in range(DH // L):
            zrow[r, pl.ds(k * L, L)] = jnp.zeros((L,), jnp.float32)
        return 0
    lax.fori_loop(0, ZR, zfill, 0)
    for k in range(RPT // L):
        zdeg[pl.ds(k * L, L)] = jnp.zeros((L,), jnp.float32)
    for k in range(ZR // L):
        ones_v[pl.ds(k * L, L)] = jnp.ones((L,), jnp.float32)

    for k in range(RPT // ZR):
        pltpu.sync_copy(zrow, acc_sh.at[pl.ds(sid * RPT + k * ZR, ZR)])

    @pl.when(cid == 0)
    def _():
        pltpu.sync_copy(zdeg, deg_sh.at[pl.ds(sid * RPT, RPT)])
    plsc.subcore_barrier()

    pltpu.sync_copy(src_hbm.at[sid], src_buf)
    pltpu.sync_copy(dst_hbm.at[sid], dst_buf)

    def edge_loop(x_hbm, with_deg):
        def start(j, buf, sem):
            pltpu.async_copy(x_hbm.at[src_buf.at[j]], buf, sem)

        def finish(j, buf, sem):
            pltpu.make_async_copy(x_hbm.at[src_buf.at[j]], buf, sem).wait()

        def consume(j, buf):
            pltpu.sync_copy(buf, acc_sh.at[dst_buf.at[j]], add=True)
            if with_deg:
                pltpu.sync_copy(ones_v.at[pl.ds(0, CH1)],
                                deg_sh.at[dst_buf.at[j]], add=True)

        start(0, rows_a, sem_a)

        def body2(i, _):
            j0 = 2 * i
            start(j0 + 1, rows_b, sem_b)
            finish(j0, rows_a, sem_a)
            consume(j0, rows_a)

            @pl.when(j0 + 2 < CPT1)
            def _():
                start(j0 + 2, rows_a, sem_a)
            finish(j0 + 1, rows_b, sem_b)
            consume(j0 + 1, rows_b)
            return 0
        lax.fori_loop(0, CPT1 // 2, body2, 0)

    @pl.when(cid == 0)
    def _():
        edge_loop(x0_hbm, True)

    @pl.when(cid == 1)
    def _():
        edge_loop(x1_hbm, False)
    plsc.subcore_barrier()

    pltpu.sync_copy(acc_sh.at[pl.ds(sid * RPT, RPT)],
                    acc_out.at[cid, pl.ds(sid * RPT, RPT)])

    @pl.when(cid == 0)
    def _():
        pltpu.sync_copy(deg_sh.at[pl.ds(sid * RPT, RPT)],
                        deg_out.at[pl.ds(sid * RPT, RPT)])


@functools.partial(
    pl.kernel,
    out_type=jax.ShapeDtypeStruct((NC, NP), jnp.float32),
    mesh=_mesh,
    scratch_types=[
        pltpu.VMEM((CPT2, CH2), jnp.int32),
        pltpu.VMEM((CPT2, CH2), jnp.int32),
        pltpu.VMEM((N,), jnp.float32),
        pltpu.VMEM((CPT2, CH2), jnp.float32),
        pltpu.VMEM((RPT,), jnp.float32),
        pltpu.VMEM_SHARED((NP,), jnp.float32),
        pltpu.SemaphoreType.DMA,
    ],
    compiler_params=pltpu.CompilerParams(use_tc_tiling_on_sc=False,
                                         needs_layout_passes=False),
)
def _sc_agg_scalar(s_hbm, src_hbm, dst_hbm, agg_out,
                   src_buf, dst_buf, s_tile, vals_all, zdeg, agg_sh, sem):
    cid = lax.axis_index("c")
    sid = lax.axis_index("s")
    w = cid * NS + sid

    for k in range(RPT // L):
        zdeg[pl.ds(k * L, L)] = jnp.zeros((L,), jnp.float32)
    pltpu.sync_copy(zdeg, agg_sh.at[pl.ds(sid * RPT, RPT)])
    plsc.subcore_barrier()

    pltpu.sync_copy(s_hbm, s_tile)
    pltpu.sync_copy(src_hbm.at[w], src_buf)
    pltpu.sync_copy(dst_hbm.at[w], dst_buf)

    def gbody(j, _):
        for k in range(CH2 // L):
            idx = src_buf[j, pl.ds(k * L, L)]
            vals_all[j, pl.ds(k * L, L)] = plsc.load_gather(s_tile, [idx])
        return 0
    lax.fori_loop(0, CPT2, gbody, 0)

    def sbody(j, _):
        pltpu.sync_copy(vals_all.at[j], agg_sh.at[dst_buf.at[j]], add=True)
        return 0
    lax.fori_loop(0, CPT2, sbody, 0)
    plsc.subcore_barrier()

    pltpu.sync_copy(agg_sh.at[pl.ds(sid * RPT, RPT)],
                    agg_out.at[cid, pl.ds(sid * RPT, RPT)])


def _tc_root_body(x_ref, w1rt_ref, b1l_ref, r_ref):
    r_ref[...] = (jnp.dot(x_ref[...], w1rt_ref[...],
                          preferred_element_type=jnp.float32) + b1l_ref[...])


_tc_root = pl.pallas_call(
    _tc_root_body,
    grid=(N // BN,),
    in_specs=[
        pl.BlockSpec((BN, D), lambda i: (i, 0)),
        pl.BlockSpec((D, D), lambda i: (0, 0)),
        pl.BlockSpec((1, D), lambda i: (0, 0)),
    ],
    out_specs=pl.BlockSpec((BN, D), lambda i: (i, 0)),
    out_shape=jax.ShapeDtypeStruct((N, D), jnp.float32),
)


def _tc_dense_body(acc_ref, deg_ref, r_ref, w1lt_ref,
                   w2lt_ref, w2rt_ref, s2l_ref, s2r_ref):
    d = jnp.maximum(deg_ref[...], 1.0)
    m0 = acc_ref[0] / d
    m1 = acc_ref[1] / d
    w1lt = w1lt_ref[...]
    h = (jnp.dot(m0, w1lt[:DH], preferred_element_type=jnp.float32)
         + jnp.dot(m1, w1lt[DH:], preferred_element_type=jnp.float32)
         + r_ref[...])
    h = jnp.maximum(h, 0.0)
    s2l_ref[...] = jnp.dot(h, w2lt_ref[...], preferred_element_type=jnp.float32)
    s2r_ref[...] = jnp.dot(h, w2rt_ref[...], preferred_element_type=jnp.float32)


_tc_dense = pl.pallas_call(
    _tc_dense_body,
    grid=(N // BN,),
    in_specs=[
        pl.BlockSpec((NC, BN, DH), lambda i: (0, i, 0)),
        pl.BlockSpec((BN, 1), lambda i: (i, 0)),
        pl.BlockSpec((BN, D), lambda i: (i, 0)),
        pl.BlockSpec((D, D), lambda i: (0, 0)),
        pl.BlockSpec((D, 1), lambda i: (0, 0)),
        pl.BlockSpec((D, 1), lambda i: (0, 0)),
    ],
    out_specs=[
        pl.BlockSpec((BN, 1), lambda i: (i, 0)),
        pl.BlockSpec((BN, 1), lambda i: (i, 0)),
    ],
    out_shape=[
        jax.ShapeDtypeStruct((N, 1), jnp.float32),
        jax.ShapeDtypeStruct((N, 1), jnp.float32),
    ],
)


def _tc_final_body(agg_ref, deg_ref, s2r_ref, b2l_ref, out_ref):
    a = agg_ref[0] + agg_ref[1]
    d = jnp.maximum(deg_ref[...], 1.0)
    out_ref[...] = a / d + b2l_ref[...] + s2r_ref[...]


_tc_final = pl.pallas_call(
    _tc_final_body,
    grid=(N // BN,),
    in_specs=[
        pl.BlockSpec((NC, BN, 1), lambda i: (0, i, 0)),
        pl.BlockSpec((BN, 1), lambda i: (i, 0)),
        pl.BlockSpec((BN, 1), lambda i: (i, 0)),
        pl.BlockSpec((1, 1), lambda i: (0, 0)),
    ],
    out_specs=pl.BlockSpec((BN, 1), lambda i: (i, 0)),
    out_shape=jax.ShapeDtypeStruct((N, 1), jnp.float32),
)


def kernel(x, edge_index, W1l, b1l, W1r, W2l, b2l, W2r):
    src_a = edge_index[0].reshape(NS, CPT1, CH1)
    dst_a = edge_index[1].reshape(NS, CPT1, CH1)
    src_b = edge_index[0].reshape(NT, CPT2, CH2)
    dst_b = edge_index[1].reshape(NT, CPT2, CH2)

    x0 = x[:, :DH]
    x1 = x[:, DH:]
    r = _tc_root(x, W1r.T, b1l.reshape(1, D))
    acc, deg = _sc_agg_rows(x0, x1, src_a, dst_a)
    deg2 = deg.reshape(NP, 1)

    s2l, s2r = _tc_dense(acc, deg2, r, W1l.T, W2l.T, W2r.T)

    agg2 = _sc_agg_scalar(s2l.reshape(N), src_b, dst_b)
    out = _tc_final(agg2.reshape(NC, NP, 1), deg2, s2r, b2l.reshape(1, 1))
    return out[:, 0]


# speedup vs baseline: 16.1110x; 1.1110x over previous
import functools

import jax
import jax.numpy as jnp
from jax import lax
from jax.experimental import pallas as pl
from jax.experimental.pallas import tpu as pltpu
from jax.experimental.pallas import tpu_sc as plsc

N = 10000
NP = 10240
D = 128
DH = D // 2
E = 320000
NC, NS, L = 2, 16, 16
NT = NC * NS
CH1 = 125
CPT1 = (E // NS) // CH1
CH2 = 80
CPT2 = (E // NS) // CH2
RPT = NP // NS
ZR = 128
BN = 2000

_mesh = plsc.VectorSubcoreMesh(core_axis_name="c", subcore_axis_name="s")
_mesh1 = plsc.VectorSubcoreMesh(core_axis_name="c", subcore_axis_name="s",
                                num_cores=1)


@functools.partial(
    pl.kernel,
    out_type=(
        jax.ShapeDtypeStruct((NC, NP, DH), jnp.float32),
        jax.ShapeDtypeStruct((NP,), jnp.float32),
    ),
    mesh=_mesh,
    scratch_types=[
        pltpu.VMEM((CPT1, CH1), jnp.int32),
        pltpu.VMEM((CPT1, CH1), jnp.int32),
        pltpu.VMEM((CH1, DH), jnp.float32),
        pltpu.VMEM((CH1, DH), jnp.float32),
        pltpu.VMEM((ZR,), jnp.float32),
        pltpu.VMEM((ZR, DH), jnp.float32),
        pltpu.VMEM((RPT,), jnp.float32),
        pltpu.VMEM_SHARED((NP, DH), jnp.float32),
        pltpu.VMEM_SHARED((NP,), jnp.float32),
        pltpu.SemaphoreType.DMA,
        pltpu.SemaphoreType.DMA,
    ],
    compiler_params=pltpu.CompilerParams(use_tc_tiling_on_sc=False),
)
def _sc_agg_rows(xr_hbm, srcA_hbm, srcB_hbm, dst_hbm, acc_out, deg_out,
                 src_buf, dst_buf, rows_a, rows_b, ones_v, zrow, zdeg,
                 acc_sh, deg_sh, sem_a, sem_b):
    cid = lax.axis_index("c")
    sid = lax.axis_index("s")

    def zfill(r, _):
        for k in range(DH // L):
            zrow[r, pl.ds(k * L, L)] = jnp.zeros((L,), jnp.float32)
        return 0
    lax.fori_loop(0, ZR, zfill, 0)
    for k in range(RPT // L):
        zdeg[pl.ds(k * L, L)] = jnp.zeros((L,), jnp.float32)
    for k in range(ZR // L):
        ones_v[pl.ds(k * L, L)] = jnp.ones((L,), jnp.float32)

    for k in range(RPT // ZR):
        pltpu.sync_copy(zrow, acc_sh.at[pl.ds(sid * RPT + k * ZR, ZR)])

    @pl.when(cid == 0)
    def _():
        pltpu.sync_copy(zdeg, deg_sh.at[pl.ds(sid * RPT, RPT)])
    plsc.subcore_barrier()

    @pl.when(cid == 0)
    def _():
        pltpu.sync_copy(srcA_hbm.at[sid], src_buf)

    @pl.when(cid == 1)
    def _():
        pltpu.sync_copy(srcB_hbm.at[sid], src_buf)
    pltpu.sync_copy(dst_hbm.at[sid], dst_buf)

    def edge_loop(with_deg):
        def start(j, buf, sem):
            pltpu.async_copy(xr_hbm.at[src_buf.at[j]], buf, sem)

        def finish(j, buf, sem):
            pltpu.make_async_copy(xr_hbm.at[src_buf.at[j]], buf, sem).wait()

        def consume(j, buf):
            pltpu.sync_copy(buf, acc_sh.at[dst_buf.at[j]], add=True)
            if with_deg:
                pltpu.sync_copy(ones_v.at[pl.ds(0, CH1)],
                                deg_sh.at[dst_buf.at[j]], add=True)

        start(0, rows_a, sem_a)

        def body2(i, _):
            j0 = 2 * i
            start(j0 + 1, rows_b, sem_b)
            finish(j0, rows_a, sem_a)
            consume(j0, rows_a)

            @pl.when(j0 + 2 < CPT1)
            def _():
                start(j0 + 2, rows_a, sem_a)
            finish(j0 + 1, rows_b, sem_b)
            consume(j0 + 1, rows_b)
            return 0
        lax.fori_loop(0, CPT1 // 2, body2, 0)

    @pl.when(cid == 0)
    def _():
        edge_loop(True)

    @pl.when(cid == 1)
    def _():
        edge_loop(False)
    plsc.subcore_barrier()

    pltpu.sync_copy(acc_sh.at[pl.ds(sid * RPT, RPT)],
                    acc_out.at[cid, pl.ds(sid * RPT, RPT)])

    @pl.when(cid == 0)
    def _():
        pltpu.sync_copy(deg_sh.at[pl.ds(sid * RPT, RPT)],
                        deg_out.at[pl.ds(sid * RPT, RPT)])


@functools.partial(
    pl.kernel,
    out_type=jax.ShapeDtypeStruct((NP,), jnp.float32),
    mesh=_mesh1,
    scratch_types=[
        pltpu.VMEM((CPT2, CH2), jnp.int32),
        pltpu.VMEM((CPT2, CH2), jnp.int32),
        pltpu.VMEM((N,), jnp.float32),
        pltpu.VMEM((CPT2, CH2), jnp.float32),
        pltpu.VMEM((RPT,), jnp.float32),
        pltpu.VMEM((RPT,), jnp.float32),
        pltpu.VMEM((RPT,), jnp.float32),
        pltpu.VMEM((RPT,), jnp.float32),
        pltpu.VMEM((RPT,), jnp.float32),
        pltpu.VMEM_SHARED((NP,), jnp.float32),
        pltpu.SemaphoreType.DMA,
    ],
    compiler_params=pltpu.CompilerParams(use_tc_tiling_on_sc=False,
                                         needs_layout_passes=False),
)
def _sc_agg_scalar(s_hbm, deg_hbm, s2rb_hbm, src_hbm, dst_hbm, out_hbm,
                   src_buf, dst_buf, s_tile, vals_all, zdeg,
                   agg_t, deg_t, s2r_t, out_t, agg_sh, sem):
    sid = lax.axis_index("s")

    for k in range(RPT // L):
        zdeg[pl.ds(k * L, L)] = jnp.zeros((L,), jnp.float32)
    pltpu.sync_copy(zdeg, agg_sh.at[pl.ds(sid * RPT, RPT)])
    plsc.subcore_barrier()

    pltpu.sync_copy(s_hbm, s_tile)
    pltpu.sync_copy(src_hbm.at[sid], src_buf)
    pltpu.sync_copy(dst_hbm.at[sid], dst_buf)

    def gbody(j, _):
        for k in range(CH2 // L):
            idx = src_buf[j, pl.ds(k * L, L)]
            vals_all[j, pl.ds(k * L, L)] = plsc.load_gather(s_tile, [idx])
        return 0
    lax.fori_loop(0, CPT2, gbody, 0)

    def sbody(j, _):
        pltpu.sync_copy(vals_all.at[j], agg_sh.at[dst_buf.at[j]], add=True)
        return 0
    lax.fori_loop(0, CPT2, sbody, 0)
    plsc.subcore_barrier()

    pltpu.sync_copy(agg_sh.at[pl.ds(sid * RPT, RPT)], agg_t)
    pltpu.sync_copy(deg_hbm.at[pl.ds(sid * RPT, RPT)], deg_t)
    pltpu.sync_copy(s2rb_hbm.at[pl.ds(sid * RPT, RPT)], s2r_t)

    def ebody(k, _):
        a = agg_t[pl.ds(k * L, L)]
        d = jnp.maximum(deg_t[pl.ds(k * L, L)], 1.0)
        out_t[pl.ds(k * L, L)] = a / d + s2r_t[pl.ds(k * L, L)]
        return 0
    lax.fori_loop(0, RPT // L, ebody, 0)
    pltpu.sync_copy(out_t, out_hbm.at[pl.ds(sid * RPT, RPT)])


def _tc_root_body(x_ref, w1rt_ref, b1l_ref, r_ref):
    r_ref[...] = (jnp.dot(x_ref[...], w1rt_ref[...],
                          preferred_element_type=jnp.float32) + b1l_ref[...])


_tc_root = pl.pallas_call(
    _tc_root_body,
    grid=(N // BN,),
    in_specs=[
        pl.BlockSpec((BN, D), lambda i: (i, 0)),
        pl.BlockSpec((D, D), lambda i: (0, 0)),
        pl.BlockSpec((1, D), lambda i: (0, 0)),
    ],
    out_specs=pl.BlockSpec((BN, D), lambda i: (i, 0)),
    out_shape=jax.ShapeDtypeStruct((N, D), jnp.float32),
)


def _tc_dense_body(acc_ref, deg_ref, r_ref, w1lt_ref,
                   w2lt_ref, w2rt_ref, b2l_ref, s2l_ref, s2r_ref):
    d = jnp.maximum(deg_ref[...], 1.0)
    m0 = acc_ref[0] / d
    m1 = acc_ref[1] / d
    w1lt = w1lt_ref[...]
    h = (jnp.dot(m0, w1lt[:DH], preferred_element_type=jnp.float32)
         + jnp.dot(m1, w1lt[DH:], preferred_element_type=jnp.float32)
         + r_ref[...])
    h = jnp.maximum(h, 0.0)
    s2l_ref[...] = jnp.dot(h, w2lt_ref[...], preferred_element_type=jnp.float32)
    s2r_ref[...] = (jnp.dot(h, w2rt_ref[...], preferred_element_type=jnp.float32)
                    + b2l_ref[...])


_tc_dense = pl.pallas_call(
    _tc_dense_body,
    grid=(N // BN,),
    in_specs=[
        pl.BlockSpec((NC, BN, DH), lambda i: (0, i, 0)),
        pl.BlockSpec((BN, 1), lambda i: (i, 0)),
        pl.BlockSpec((BN, D), lambda i: (i, 0)),
        pl.BlockSpec((D, D), lambda i: (0, 0)),
        pl.BlockSpec((D, 1), lambda i: (0, 0)),
        pl.BlockSpec((D, 1), lambda i: (0, 0)),
        pl.BlockSpec((1, 1), lambda i: (0, 0)),
    ],
    out_specs=[
        pl.BlockSpec((BN, 1), lambda i: (i, 0)),
        pl.BlockSpec((BN, 1), lambda i: (i, 0)),
    ],
    out_shape=[
        jax.ShapeDtypeStruct((N, 1), jnp.float32),
        jax.ShapeDtypeStruct((NP, 1), jnp.float32),
    ],
)


def kernel(x, edge_index, W1l, b1l, W1r, W2l, b2l, W2r):
    src = edge_index[0]
    srcA = (src * 2).reshape(NS, CPT1, CH1)
    srcB = (src * 2 + 1).reshape(NS, CPT1, CH1)
    dst_a = edge_index[1].reshape(NS, CPT1, CH1)
    src_c = src.reshape(NS, CPT2, CH2)
    dst_c = edge_index[1].reshape(NS, CPT2, CH2)

    xr = x.reshape(2 * N, DH)
    r = _tc_root(x, W1r.T, b1l.reshape(1, D))
    acc, deg = _sc_agg_rows(xr, srcA, srcB, dst_a)
    deg2 = deg.reshape(NP, 1)

    s2l, s2rb = _tc_dense(acc, deg2, r, W1l.T, W2l.T, W2r.T,
                          b2l.reshape(1, 1))

    out = _sc_agg_scalar(s2l.reshape(N), deg, s2rb.reshape(NP), src_c, dst_c)
    return out[:N]


# speedup vs baseline: 18.9802x; 1.1781x over previous
import functools

import jax
import jax.numpy as jnp
from jax import lax
from jax.experimental import pallas as pl
from jax.experimental.pallas import tpu as pltpu
from jax.experimental.pallas import tpu_sc as plsc

N = 10000
NP = 10240
D = 128
DH = D // 2
E = 320000
NC, NS, L = 2, 16, 16
NT = NC * NS
CH1 = 125
CPT1 = (E // NS) // CH1
CH2 = 80
CPT2 = (E // NS) // CH2
RPT = NP // NS
ZR = 128
BN = 2000
NB1 = 4
NB2 = 5

_mesh = plsc.VectorSubcoreMesh(core_axis_name="c", subcore_axis_name="s")
_mesh1 = plsc.VectorSubcoreMesh(core_axis_name="c", subcore_axis_name="s",
                                num_cores=1)


@functools.partial(
    pl.kernel,
    out_type=(
        jax.ShapeDtypeStruct((NC, NP, DH), jnp.float32),
        jax.ShapeDtypeStruct((NC, NP), jnp.float32),
    ),
    mesh=_mesh,
    scratch_types=[
        pltpu.VMEM((CPT1, CH1), jnp.int32),
        pltpu.VMEM((CPT1, CH1), jnp.int32),
    ] + [pltpu.VMEM((CH1, DH), jnp.float32)] * NB1 + [
        pltpu.VMEM((ZR,), jnp.float32),
        pltpu.VMEM((ZR, DH), jnp.float32),
        pltpu.VMEM((RPT,), jnp.float32),
        pltpu.VMEM_SHARED((NP, DH), jnp.float32),
        pltpu.VMEM_SHARED((NP,), jnp.float32),
    ] + [pltpu.SemaphoreType.DMA] * NB1,
    compiler_params=pltpu.CompilerParams(use_tc_tiling_on_sc=False),
)
def _sc_agg_rows(xr_hbm, srcA_hbm, srcB_hbm, dst_hbm, acc_out, deg_out,
                 src_buf, dst_buf, rows_0, rows_1, rows_2, rows_3,
                 ones_v, zrow, zdeg, acc_sh, deg_sh,
                 sem_0, sem_1, sem_2, sem_3):
    cid = lax.axis_index("c")
    sid = lax.axis_index("s")
    bufs = [rows_0, rows_1, rows_2, rows_3]
    sems = [sem_0, sem_1, sem_2, sem_3]

    def zfill(r, _):
        for k in range(DH // L):
            zrow[r, pl.ds(k * L, L)] = jnp.zeros((L,), jnp.float32)
        return 0
    lax.fori_loop(0, ZR, zfill, 0)
    for k in range(RPT // L):
        zdeg[pl.ds(k * L, L)] = jnp.zeros((L,), jnp.float32)
    for k in range(ZR // L):
        ones_v[pl.ds(k * L, L)] = jnp.ones((L,), jnp.float32)

    for k in range(RPT // ZR):
        pltpu.sync_copy(zrow, acc_sh.at[pl.ds(sid * RPT + k * ZR, ZR)])
    pltpu.sync_copy(zdeg, deg_sh.at[pl.ds(sid * RPT, RPT)])
    plsc.subcore_barrier()

    @pl.when(cid == 0)
    def _():
        pltpu.sync_copy(srcA_hbm.at[sid], src_buf)

    @pl.when(cid == 1)
    def _():
        pltpu.sync_copy(srcB_hbm.at[sid], src_buf)
    pltpu.sync_copy(dst_hbm.at[sid], dst_buf)

    def start(j, b):
        pltpu.async_copy(xr_hbm.at[src_buf.at[j]], bufs[b], sems[b])

    def finish(j, b):
        pltpu.make_async_copy(xr_hbm.at[src_buf.at[j]], bufs[b],
                              sems[b]).wait()

    for k in range(NB1 - 1):
        start(k, k)

    def body(i, _):
        for k in range(NB1):
            j = NB1 * i + k

            @pl.when(j + NB1 - 1 < CPT1)
            def _():
                start(j + NB1 - 1, (k + NB1 - 1) % NB1)
            finish(j, k)
            pltpu.sync_copy(bufs[k], acc_sh.at[dst_buf.at[j]], add=True)

            @pl.when(cid == (k % 2))
            def _():
                pltpu.sync_copy(ones_v.at[pl.ds(0, CH1)],
                                deg_sh.at[dst_buf.at[j]], add=True)
        return 0
    lax.fori_loop(0, CPT1 // NB1, body, 0)
    plsc.subcore_barrier()

    pltpu.sync_copy(acc_sh.at[pl.ds(sid * RPT, RPT)],
                    acc_out.at[cid, pl.ds(sid * RPT, RPT)])
    pltpu.sync_copy(deg_sh.at[pl.ds(sid * RPT, RPT)],
                    deg_out.at[cid, pl.ds(sid * RPT, RPT)])


@functools.partial(
    pl.kernel,
    out_type=jax.ShapeDtypeStruct((NP,), jnp.float32),
    mesh=_mesh1,
    scratch_types=[
        pltpu.VMEM((CPT2, CH2), jnp.int32),
        pltpu.VMEM((CPT2, CH2), jnp.int32),
        pltpu.VMEM((N,), jnp.float32),
        pltpu.VMEM((CPT2, CH2), jnp.float32),
        pltpu.VMEM((RPT,), jnp.float32),
        pltpu.VMEM((RPT,), jnp.float32),
        pltpu.VMEM((RPT,), jnp.float32),
        pltpu.VMEM((RPT,), jnp.float32),
        pltpu.VMEM((RPT,), jnp.float32),
        pltpu.VMEM_SHARED((NP,), jnp.float32),
    ] + [pltpu.SemaphoreType.DMA] * NB2,
    compiler_params=pltpu.CompilerParams(use_tc_tiling_on_sc=False,
                                         needs_layout_passes=False),
)
def _sc_agg_scalar(s_hbm, degc_hbm, s2rb_hbm, src_hbm, dst_hbm, out_hbm,
                   src_buf, dst_buf, s_tile, vals_all, zdeg,
                   agg_t, deg_t, s2r_t, out_t, agg_sh,
                   sem_0, sem_1, sem_2, sem_3, sem_4):
    sid = lax.axis_index("s")
    sems = [sem_0, sem_1, sem_2, sem_3, sem_4]

    for k in range(RPT // L):
        zdeg[pl.ds(k * L, L)] = jnp.zeros((L,), jnp.float32)
    pltpu.sync_copy(zdeg, agg_sh.at[pl.ds(sid * RPT, RPT)])
    plsc.subcore_barrier()

    pltpu.sync_copy(s_hbm, s_tile)
    pltpu.sync_copy(src_hbm.at[sid], src_buf)
    pltpu.sync_copy(dst_hbm.at[sid], dst_buf)

    def gbody(j, _):
        for k in range(CH2 // L):
            idx = src_buf[j, pl.ds(k * L, L)]
            vals_all[j, pl.ds(k * L, L)] = plsc.load_gather(s_tile, [idx])
        return 0
    lax.fori_loop(0, CPT2, gbody, 0)

    def sstart(j, b):
        pltpu.async_copy(vals_all.at[j], agg_sh.at[dst_buf.at[j]], sems[b],
                         add=True)

    def sfinish(j, b):
        pltpu.make_async_copy(vals_all.at[j], agg_sh.at[dst_buf.at[j]],
                              sems[b]).wait()

    def sbody(i, _):
        for k in range(NB2):
            j = NB2 * i + k

            @pl.when(j >= NB2)
            def _():
                sfinish(j - NB2, k)
            sstart(j, k)
        return 0
    lax.fori_loop(0, CPT2 // NB2, sbody, 0)
    for k in range(NB2):
        sfinish(CPT2 - NB2 + k, k)
    plsc.subcore_barrier()

    pltpu.sync_copy(agg_sh.at[pl.ds(sid * RPT, RPT)], agg_t)
    pltpu.sync_copy(degc_hbm.at[pl.ds(sid * RPT, RPT)], deg_t)
    pltpu.sync_copy(s2rb_hbm.at[pl.ds(sid * RPT, RPT)], s2r_t)

    def ebody(k, _):
        a = agg_t[pl.ds(k * L, L)]
        d = deg_t[pl.ds(k * L, L)]
        out_t[pl.ds(k * L, L)] = a / d + s2r_t[pl.ds(k * L, L)]
        return 0
    lax.fori_loop(0, RPT // L, ebody, 0)
    pltpu.sync_copy(out_t, out_hbm.at[pl.ds(sid * RPT, RPT)])


def _tc_root_body(x_ref, w1rt_ref, b1l_ref, r_ref):
    r_ref[...] = (jnp.dot(x_ref[...], w1rt_ref[...],
                          preferred_element_type=jnp.float32) + b1l_ref[...])


_tc_root = pl.pallas_call(
    _tc_root_body,
    grid=(N // BN,),
    in_specs=[
        pl.BlockSpec((BN, D), lambda i: (i, 0)),
        pl.BlockSpec((D, D), lambda i: (0, 0)),
        pl.BlockSpec((1, D), lambda i: (0, 0)),
    ],
    out_specs=pl.BlockSpec((BN, D), lambda i: (i, 0)),
    out_shape=jax.ShapeDtypeStruct((N, D), jnp.float32),
)


def _tc_dense_body(acc_ref, deg_ref, r_ref, w1lt_ref,
                   w2lt_ref, w2rt_ref, b2l_ref, s2l_ref, s2r_ref, degc_ref):
    d = jnp.maximum(deg_ref[0] + deg_ref[1], 1.0)
    degc_ref[...] = d
    m0 = acc_ref[0] / d
    m1 = acc_ref[1] / d
    w1lt = w1lt_ref[...]
    h = (jnp.dot(m0, w1lt[:DH], preferred_element_type=jnp.float32)
         + jnp.dot(m1, w1lt[DH:], preferred_element_type=jnp.float32)
         + r_ref[...])
    h = jnp.maximum(h, 0.0)
    s2l_ref[...] = jnp.dot(h, w2lt_ref[...], preferred_element_type=jnp.float32)
    s2r_ref[...] = (jnp.dot(h, w2rt_ref[...], preferred_element_type=jnp.float32)
                    + b2l_ref[...])


_tc_dense = pl.pallas_call(
    _tc_dense_body,
    grid=(N // BN,),
    in_specs=[
        pl.BlockSpec((NC, BN, DH), lambda i: (0, i, 0)),
        pl.BlockSpec((NC, BN, 1), lambda i: (0, i, 0)),
        pl.BlockSpec((BN, D), lambda i: (i, 0)),
        pl.BlockSpec((D, D), lambda i: (0, 0)),
        pl.BlockSpec((D, 1), lambda i: (0, 0)),
        pl.BlockSpec((D, 1), lambda i: (0, 0)),
        pl.BlockSpec((1, 1), lambda i: (0, 0)),
    ],
    out_specs=[
        pl.BlockSpec((BN, 1), lambda i: (i, 0)),
        pl.BlockSpec((BN, 1), lambda i: (i, 0)),
        pl.BlockSpec((BN, 1), lambda i: (i, 0)),
    ],
    out_shape=[
        jax.ShapeDtypeStruct((N, 1), jnp.float32),
        jax.ShapeDtypeStruct((NP, 1), jnp.float32),
        jax.ShapeDtypeStruct((NP, 1), jnp.float32),
    ],
)


def kernel(x, edge_index, W1l, b1l, W1r, W2l, b2l, W2r):
    src = edge_index[0]
    srcA = (src * 2).reshape(NS, CPT1, CH1)
    srcB = (src * 2 + 1).reshape(NS, CPT1, CH1)
    dst_a = edge_index[1].reshape(NS, CPT1, CH1)
    src_c = src.reshape(NS, CPT2, CH2)
    dst_c = edge_index[1].reshape(NS, CPT2, CH2)

    xr = x.reshape(2 * N, DH)
    r = _tc_root(x, W1r.T, b1l.reshape(1, D))
    acc, deg = _sc_agg_rows(xr, srcA, srcB, dst_a)

    s2l, s2rb, degc = _tc_dense(acc, deg.reshape(NC, NP, 1), r,
                                W1l.T, W2l.T, W2r.T, b2l.reshape(1, 1))

    out = _sc_agg_scalar(s2l.reshape(N), degc.reshape(NP),
                         s2rb.reshape(NP), src_c, dst_c)
    return out[:N]


# speedup vs baseline: 19.4923x; 1.0270x over previous
import functools

import jax
import jax.numpy as jnp
from jax import lax
from jax.experimental import pallas as pl
from jax.experimental.pallas import tpu as pltpu
from jax.experimental.pallas import tpu_sc as plsc

N = 10000
NP = 10240
D = 128
DH = D // 2
E = 320000
NC, NS, L = 2, 16, 16
NT = NC * NS
CH1 = 125
CPT1 = (E // NS) // CH1
CH2 = 80
CPT2 = (E // NS) // CH2
RPT = NP // NS
ZR = 128
BN = 2000
NB1 = 4
NB2 = 5

_mesh = plsc.VectorSubcoreMesh(core_axis_name="c", subcore_axis_name="s")
_mesh1 = plsc.VectorSubcoreMesh(core_axis_name="c", subcore_axis_name="s",
                                num_cores=1)


@functools.partial(
    pl.kernel,
    out_type=(
        jax.ShapeDtypeStruct((NC, NP, DH), jnp.float32),
        jax.ShapeDtypeStruct((NC, NP), jnp.float32),
    ),
    mesh=_mesh,
    scratch_types=[
        pltpu.VMEM((CPT1, CH1), jnp.int32),
        pltpu.VMEM((CPT1, CH1), jnp.int32),
    ] + [pltpu.VMEM((CH1, DH), jnp.float32)] * NB1 + [
        pltpu.VMEM((ZR,), jnp.float32),
        pltpu.VMEM((ZR, DH), jnp.float32),
        pltpu.VMEM((RPT,), jnp.float32),
        pltpu.VMEM_SHARED((NP, DH), jnp.float32),
        pltpu.VMEM_SHARED((NP,), jnp.float32),
    ] + [pltpu.SemaphoreType.DMA] * NB1,
    compiler_params=pltpu.CompilerParams(use_tc_tiling_on_sc=False),
)
def _sc_agg_rows(xr_hbm, srcA_hbm, srcB_hbm, dst_hbm, acc_out, deg_out,
                 src_buf, dst_buf, rows_0, rows_1, rows_2, rows_3,
                 ones_v, zrow, zdeg, acc_sh, deg_sh,
                 sem_0, sem_1, sem_2, sem_3):
    cid = lax.axis_index("c")
    sid = lax.axis_index("s")
    bufs = [rows_0, rows_1, rows_2, rows_3]
    sems = [sem_0, sem_1, sem_2, sem_3]

    def zfill(r, _):
        for k in range(DH // L):
            zrow[r, pl.ds(k * L, L)] = jnp.zeros((L,), jnp.float32)
        return 0
    lax.fori_loop(0, ZR, zfill, 0)
    for k in range(RPT // L):
        zdeg[pl.ds(k * L, L)] = jnp.zeros((L,), jnp.float32)
    for k in range(ZR // L):
        ones_v[pl.ds(k * L, L)] = jnp.ones((L,), jnp.float32)

    for k in range(RPT // ZR):
        pltpu.sync_copy(zrow, acc_sh.at[pl.ds(sid * RPT + k * ZR, ZR)])
    pltpu.sync_copy(zdeg, deg_sh.at[pl.ds(sid * RPT, RPT)])
    plsc.subcore_barrier()

    @pl.when(cid == 0)
    def _():
        pltpu.sync_copy(srcA_hbm.at[sid], src_buf)

    @pl.when(cid == 1)
    def _():
        pltpu.sync_copy(srcB_hbm.at[sid], src_buf)
    pltpu.sync_copy(dst_hbm.at[sid], dst_buf)

    def start(j, b):
        pltpu.async_copy(xr_hbm.at[src_buf.at[j]], bufs[b], sems[b])

    def finish(j, b):
        pltpu.make_async_copy(xr_hbm.at[src_buf.at[j]], bufs[b],
                              sems[b]).wait()

    for k in range(NB1 - 1):
        start(k, k)

    def body(i, _):
        for k in range(NB1):
            j = NB1 * i + k

            @pl.when(j + NB1 - 1 < CPT1)
            def _():
                start(j + NB1 - 1, (k + NB1 - 1) % NB1)
            finish(j, k)
            pltpu.sync_copy(bufs[k], acc_sh.at[dst_buf.at[j]], add=True)

            @pl.when(cid == (k % 2))
            def _():
                pltpu.sync_copy(ones_v.at[pl.ds(0, CH1)],
                                deg_sh.at[dst_buf.at[j]], add=True)
        return 0
    lax.fori_loop(0, CPT1 // NB1, body, 0)
    plsc.subcore_barrier()

    pltpu.sync_copy(acc_sh.at[pl.ds(sid * RPT, RPT)],
                    acc_out.at[cid, pl.ds(sid * RPT, RPT)])
    pltpu.sync_copy(deg_sh.at[pl.ds(sid * RPT, RPT)],
                    deg_out.at[cid, pl.ds(sid * RPT, RPT)])


@functools.partial(
    pl.kernel,
    out_type=jax.ShapeDtypeStruct((NP,), jnp.float32),
    mesh=_mesh1,
    scratch_types=[
        pltpu.VMEM((CPT2, CH2), jnp.int32),
        pltpu.VMEM((CPT2, CH2), jnp.int32),
        pltpu.VMEM((N,), jnp.float32),
        pltpu.VMEM((CPT2, CH2), jnp.float32),
        pltpu.VMEM((RPT,), jnp.float32),
        pltpu.VMEM((RPT,), jnp.float32),
        pltpu.VMEM((RPT,), jnp.float32),
        pltpu.VMEM((RPT,), jnp.float32),
        pltpu.VMEM((RPT,), jnp.float32),
        pltpu.VMEM_SHARED((NP,), jnp.float32),
    ] + [pltpu.SemaphoreType.DMA] * NB2,
    compiler_params=pltpu.CompilerParams(use_tc_tiling_on_sc=False,
                                         needs_layout_passes=False),
)
def _sc_agg_scalar(s_hbm, degc_hbm, s2rb_hbm, src_hbm, dst_hbm, out_hbm,
                   src_buf, dst_buf, s_tile, vals_all, zdeg,
                   agg_t, deg_t, s2r_t, out_t, agg_sh,
                   sem_0, sem_1, sem_2, sem_3, sem_4):
    sid = lax.axis_index("s")
    sems = [sem_0, sem_1, sem_2, sem_3, sem_4]

    for k in range(RPT // L):
        zdeg[pl.ds(k * L, L)] = jnp.zeros((L,), jnp.float32)
    pltpu.sync_copy(zdeg, agg_sh.at[pl.ds(sid * RPT, RPT)])
    plsc.subcore_barrier()

    pltpu.sync_copy(s_hbm, s_tile)
    pltpu.sync_copy(src_hbm.at[sid], src_buf)
    pltpu.sync_copy(dst_hbm.at[sid], dst_buf)

    def sstart(j, b):
        pltpu.async_copy(vals_all.at[j], agg_sh.at[dst_buf.at[j]], sems[b],
                         add=True)

    def sfinish(j, b):
        pltpu.make_async_copy(vals_all.at[j], agg_sh.at[dst_buf.at[j]],
                              sems[b]).wait()

    def sbody(i, _):
        for k in range(NB2):
            j = NB2 * i + k
            for g in range(CH2 // L):
                idx = src_buf[j, pl.ds(g * L, L)]
                vals_all[j, pl.ds(g * L, L)] = plsc.load_gather(s_tile, [idx])

            @pl.when(j >= NB2)
            def _():
                sfinish(j - NB2, k)
            sstart(j, k)
        return 0
    lax.fori_loop(0, CPT2 // NB2, sbody, 0)
    for k in range(NB2):
        sfinish(CPT2 - NB2 + k, k)
    plsc.subcore_barrier()

    pltpu.sync_copy(agg_sh.at[pl.ds(sid * RPT, RPT)], agg_t)
    pltpu.sync_copy(degc_hbm.at[pl.ds(sid * RPT, RPT)], deg_t)
    pltpu.sync_copy(s2rb_hbm.at[pl.ds(sid * RPT, RPT)], s2r_t)

    def ebody(k, _):
        a = agg_t[pl.ds(k * L, L)]
        d = deg_t[pl.ds(k * L, L)]
        out_t[pl.ds(k * L, L)] = a / d + s2r_t[pl.ds(k * L, L)]
        return 0
    lax.fori_loop(0, RPT // L, ebody, 0)
    pltpu.sync_copy(out_t, out_hbm.at[pl.ds(sid * RPT, RPT)])


def _tc_dense_body(acc_ref, deg_ref, x_ref, w1lt_ref, w1rt_ref, b1l_ref,
                   w2lt_ref, w2rt_ref, b2l_ref, s2l_ref, s2r_ref, degc_ref):
    d = jnp.maximum(deg_ref[0] + deg_ref[1], 1.0)
    degc_ref[...] = d
    m0 = acc_ref[0] / d
    m1 = acc_ref[1] / d
    w1lt = w1lt_ref[...]
    h = (jnp.dot(m0, w1lt[:DH], preferred_element_type=jnp.float32)
         + jnp.dot(m1, w1lt[DH:], preferred_element_type=jnp.float32)
         + jnp.dot(x_ref[...], w1rt_ref[...], preferred_element_type=jnp.float32)
         + b1l_ref[...])
    h = jnp.maximum(h, 0.0)
    s2l_ref[...] = jnp.dot(h, w2lt_ref[...], preferred_element_type=jnp.float32)
    s2r_ref[...] = (jnp.dot(h, w2rt_ref[...], preferred_element_type=jnp.float32)
                    + b2l_ref[...])


_tc_dense = pl.pallas_call(
    _tc_dense_body,
    grid=(N // BN,),
    in_specs=[
        pl.BlockSpec((NC, BN, DH), lambda i: (0, i, 0)),
        pl.BlockSpec((NC, BN, 1), lambda i: (0, i, 0)),
        pl.BlockSpec((BN, D), lambda i: (i, 0)),
        pl.BlockSpec((D, D), lambda i: (0, 0)),
        pl.BlockSpec((D, D), lambda i: (0, 0)),
        pl.BlockSpec((1, D), lambda i: (0, 0)),
        pl.BlockSpec((D, 1), lambda i: (0, 0)),
        pl.BlockSpec((D, 1), lambda i: (0, 0)),
        pl.BlockSpec((1, 1), lambda i: (0, 0)),
    ],
    out_specs=[
        pl.BlockSpec((BN, 1), lambda i: (i, 0)),
        pl.BlockSpec((BN, 1), lambda i: (i, 0)),
        pl.BlockSpec((BN, 1), lambda i: (i, 0)),
    ],
    out_shape=[
        jax.ShapeDtypeStruct((N, 1), jnp.float32),
        jax.ShapeDtypeStruct((NP, 1), jnp.float32),
        jax.ShapeDtypeStruct((NP, 1), jnp.float32),
    ],
)


def kernel(x, edge_index, W1l, b1l, W1r, W2l, b2l, W2r):
    src = edge_index[0]
    srcA = (src * 2).reshape(NS, CPT1, CH1)
    srcB = (src * 2 + 1).reshape(NS, CPT1, CH1)
    dst_a = edge_index[1].reshape(NS, CPT1, CH1)
    src_c = src.reshape(NS, CPT2, CH2)
    dst_c = edge_index[1].reshape(NS, CPT2, CH2)

    xr = x.reshape(2 * N, DH)
    acc, deg = _sc_agg_rows(xr, srcA, srcB, dst_a)

    s2l, s2rb, degc = _tc_dense(acc, deg.reshape(NC, NP, 1), x,
                                W1l.T, W1r.T, b1l.reshape(1, D),
                                W2l.T, W2r.T, b2l.reshape(1, 1))

    out = _sc_agg_scalar(s2l.reshape(N), degc.reshape(NP),
                         s2rb.reshape(NP), src_c, dst_c)
    return out[:N]


# speedup vs baseline: 21.9391x; 1.1255x over previous
import functools

import jax
import jax.numpy as jnp
from jax import lax
from jax.experimental import pallas as pl
from jax.experimental.pallas import tpu as pltpu
from jax.experimental.pallas import tpu_sc as plsc

N = 10000
NP = 10240
D = 128
DH = D // 2
E = 320000
NC, NS, L = 2, 16, 16
NT = NC * NS
CH1 = 125
CPT1 = (E // NS) // CH1
CH2 = 80
CPT2 = (E // NS) // CH2
RPT = NP // NS
ZR = 128
BN = 1024
NB1 = 4
NB2 = 5

_mesh = plsc.VectorSubcoreMesh(core_axis_name="c", subcore_axis_name="s")
_mesh1 = plsc.VectorSubcoreMesh(core_axis_name="c", subcore_axis_name="s",
                                num_cores=1)


@functools.partial(
    pl.kernel,
    out_type=(
        jax.ShapeDtypeStruct((NC, NP, DH), jnp.float32),
        jax.ShapeDtypeStruct((NC, NP), jnp.float32),
    ),
    mesh=_mesh,
    scratch_types=[
        pltpu.VMEM((CPT1, CH1), jnp.int32),
        pltpu.VMEM((CPT1, CH1), jnp.int32),
    ] + [pltpu.VMEM((CH1, DH), jnp.float32)] * NB1 + [
        pltpu.VMEM((ZR,), jnp.float32),
        pltpu.VMEM((ZR, DH), jnp.float32),
        pltpu.VMEM((RPT,), jnp.float32),
        pltpu.VMEM_SHARED((NP, DH), jnp.float32),
        pltpu.VMEM_SHARED((NP,), jnp.float32),
    ] + [pltpu.SemaphoreType.DMA] * NB1,
    compiler_params=pltpu.CompilerParams(use_tc_tiling_on_sc=False),
)
def _sc_agg_rows(xr_hbm, srcA_hbm, srcB_hbm, dst_hbm, acc_out, deg_out,
                 src_buf, dst_buf, rows_0, rows_1, rows_2, rows_3,
                 ones_v, zrow, zdeg, acc_sh, deg_sh,
                 sem_0, sem_1, sem_2, sem_3):
    cid = lax.axis_index("c")
    sid = lax.axis_index("s")
    bufs = [rows_0, rows_1, rows_2, rows_3]
    sems = [sem_0, sem_1, sem_2, sem_3]

    def zfill(r, _):
        for k in range(DH // L):
            zrow[r, pl.ds(k * L, L)] = jnp.zeros((L,), jnp.float32)
        return 0
    lax.fori_loop(0, ZR, zfill, 0)
    for k in range(RPT // L):
        zdeg[pl.ds(k * L, L)] = jnp.zeros((L,), jnp.float32)
    for k in range(ZR // L):
        ones_v[pl.ds(k * L, L)] = jnp.ones((L,), jnp.float32)

    for k in range(RPT // ZR):
        pltpu.sync_copy(zrow, acc_sh.at[pl.ds(sid * RPT + k * ZR, ZR)])
    pltpu.sync_copy(zdeg, deg_sh.at[pl.ds(sid * RPT, RPT)])
    plsc.subcore_barrier()

    @pl.when(cid == 0)
    def _():
        pltpu.sync_copy(srcA_hbm.at[sid], src_buf)

    @pl.when(cid == 1)
    def _():
        pltpu.sync_copy(srcB_hbm.at[sid], src_buf)
    pltpu.sync_copy(dst_hbm.at[sid], dst_buf)

    def start(j, b):
        pltpu.async_copy(xr_hbm.at[src_buf.at[j]], bufs[b], sems[b])

    def finish(j, b):
        pltpu.make_async_copy(xr_hbm.at[src_buf.at[j]], bufs[b],
                              sems[b]).wait()

    for k in range(NB1 - 1):
        start(k, k)

    def body(i, _):
        for k in range(NB1):
            j = NB1 * i + k

            @pl.when(j + NB1 - 1 < CPT1)
            def _():
                start(j + NB1 - 1, (k + NB1 - 1) % NB1)
            finish(j, k)
            pltpu.sync_copy(bufs[k], acc_sh.at[dst_buf.at[j]], add=True)

            @pl.when(cid == (k % 2))
            def _():
                pltpu.sync_copy(ones_v.at[pl.ds(0, CH1)],
                                deg_sh.at[dst_buf.at[j]], add=True)
        return 0
    lax.fori_loop(0, CPT1 // NB1, body, 0)
    plsc.subcore_barrier()

    pltpu.sync_copy(acc_sh.at[pl.ds(sid * RPT, RPT)],
                    acc_out.at[cid, pl.ds(sid * RPT, RPT)])
    pltpu.sync_copy(deg_sh.at[pl.ds(sid * RPT, RPT)],
                    deg_out.at[cid, pl.ds(sid * RPT, RPT)])


@functools.partial(
    pl.kernel,
    out_type=jax.ShapeDtypeStruct((NP,), jnp.float32),
    mesh=_mesh1,
    scratch_types=[
        pltpu.VMEM((CPT2, CH2), jnp.int32),
        pltpu.VMEM((CPT2, CH2), jnp.int32),
        pltpu.VMEM((NP,), jnp.float32),
        pltpu.VMEM((CPT2, CH2), jnp.float32),
        pltpu.VMEM((RPT,), jnp.float32),
        pltpu.VMEM((RPT,), jnp.float32),
        pltpu.VMEM((RPT,), jnp.float32),
        pltpu.VMEM((RPT,), jnp.float32),
        pltpu.VMEM((RPT,), jnp.float32),
        pltpu.VMEM_SHARED((NP,), jnp.float32),
    ] + [pltpu.SemaphoreType.DMA] * NB2,
    compiler_params=pltpu.CompilerParams(use_tc_tiling_on_sc=False,
                                         needs_layout_passes=False),
)
def _sc_agg_scalar(s_hbm, degc_hbm, s2rb_hbm, src_hbm, dst_hbm, out_hbm,
                   src_buf, dst_buf, s_tile, vals_all, zdeg,
                   agg_t, deg_t, s2r_t, out_t, agg_sh,
                   sem_0, sem_1, sem_2, sem_3, sem_4):
    sid = lax.axis_index("s")
    sems = [sem_0, sem_1, sem_2, sem_3, sem_4]

    for k in range(RPT // L):
        zdeg[pl.ds(k * L, L)] = jnp.zeros((L,), jnp.float32)
    pltpu.sync_copy(zdeg, agg_sh.at[pl.ds(sid * RPT, RPT)])
    plsc.subcore_barrier()

    pltpu.sync_copy(s_hbm, s_tile)
    pltpu.sync_copy(src_hbm.at[sid], src_buf)
    pltpu.sync_copy(dst_hbm.at[sid], dst_buf)

    def sstart(j, b):
        pltpu.async_copy(vals_all.at[j], agg_sh.at[dst_buf.at[j]], sems[b],
                         add=True)

    def sfinish(j, b):
        pltpu.make_async_copy(vals_all.at[j], agg_sh.at[dst_buf.at[j]],
                              sems[b]).wait()

    def sbody(i, _):
        for k in range(NB2):
            j = NB2 * i + k
            for g in range(CH2 // L):
                idx = src_buf[j, pl.ds(g * L, L)]
                vals_all[j, pl.ds(g * L, L)] = plsc.load_gather(s_tile, [idx])

            @pl.when(j >= NB2)
            def _():
                sfinish(j - NB2, k)
            sstart(j, k)
        return 0
    lax.fori_loop(0, CPT2 // NB2, sbody, 0)
    for k in range(NB2):
        sfinish(CPT2 - NB2 + k, k)
    plsc.subcore_barrier()

    pltpu.sync_copy(agg_sh.at[pl.ds(sid * RPT, RPT)], agg_t)
    pltpu.sync_copy(degc_hbm.at[pl.ds(sid * RPT, RPT)], deg_t)
    pltpu.sync_copy(s2rb_hbm.at[pl.ds(sid * RPT, RPT)], s2r_t)

    def ebody(k, _):
        a = agg_t[pl.ds(k * L, L)]
        d = deg_t[pl.ds(k * L, L)]
        out_t[pl.ds(k * L, L)] = a / d + s2r_t[pl.ds(k * L, L)]
        return 0
    lax.fori_loop(0, RPT // L, ebody, 0)
    pltpu.sync_copy(out_t, out_hbm.at[pl.ds(sid * RPT, RPT)])


def _tc_dense_body(acc_ref, deg_ref, x_ref, w1lt_ref, w1rt_ref, b1l_ref,
                   w2lt_ref, w2rt_ref, b2l_ref, s2l_ref, s2r_ref, degc_ref):
    d = jnp.maximum(deg_ref[0] + deg_ref[1], 1.0)
    degc_ref[...] = d
    m0 = acc_ref[0] / d[:, None]
    m1 = acc_ref[1] / d[:, None]
    w1lt = w1lt_ref[...]
    h = (jnp.dot(m0, w1lt[:DH], preferred_element_type=jnp.float32)
         + jnp.dot(m1, w1lt[DH:], preferred_element_type=jnp.float32)
         + jnp.dot(x_ref[...], w1rt_ref[...], preferred_element_type=jnp.float32)
         + b1l_ref[...])
    h = jnp.maximum(h, 0.0)
    s2l_ref[...] = jnp.dot(h, w2lt_ref[...],
                           preferred_element_type=jnp.float32)[:, 0]
    s2r_ref[...] = (jnp.dot(h, w2rt_ref[...],
                            preferred_element_type=jnp.float32)[:, 0]
                    + b2l_ref[0, 0])


_tc_dense = pl.pallas_call(
    _tc_dense_body,
    grid=(NP // BN,),
    in_specs=[
        pl.BlockSpec((NC, BN, DH), lambda i: (0, i, 0)),
        pl.BlockSpec((NC, BN), lambda i: (0, i)),
        pl.BlockSpec((BN, D), lambda i: (i, 0)),
        pl.BlockSpec((D, D), lambda i: (0, 0)),
        pl.BlockSpec((D, D), lambda i: (0, 0)),
        pl.BlockSpec((1, D), lambda i: (0, 0)),
        pl.BlockSpec((D, 1), lambda i: (0, 0)),
        pl.BlockSpec((D, 1), lambda i: (0, 0)),
        pl.BlockSpec((1, 1), lambda i: (0, 0)),
    ],
    out_specs=[
        pl.BlockSpec((BN,), lambda i: (i,)),
        pl.BlockSpec((BN,), lambda i: (i,)),
        pl.BlockSpec((BN,), lambda i: (i,)),
    ],
    out_shape=[
        jax.ShapeDtypeStruct((NP,), jnp.float32),
        jax.ShapeDtypeStruct((NP,), jnp.float32),
        jax.ShapeDtypeStruct((NP,), jnp.float32),
    ],
)


def kernel(x, edge_index, W1l, b1l, W1r, W2l, b2l, W2r):
    src = edge_index[0]
    srcA = (src * 2).reshape(NS, CPT1, CH1)
    srcB = (src * 2 + 1).reshape(NS, CPT1, CH1)
    dst_a = edge_index[1].reshape(NS, CPT1, CH1)
    src_c = src.reshape(NS, CPT2, CH2)
    dst_c = edge_index[1].reshape(NS, CPT2, CH2)

    xr = x.reshape(2 * N, DH)
    acc, deg = _sc_agg_rows(xr, srcA, srcB, dst_a)

    s2l, s2rb, degc = _tc_dense(acc, deg, x,
                                W1l.T, W1r.T, b1l.reshape(1, D),
                                W2l.T, W2r.T, b2l.reshape(1, 1))

    out = _sc_agg_scalar(s2l, degc, s2rb, src_c, dst_c)
    return out[:N]
